# baseline (speedup 1.0000x reference)
import jax
import jax.numpy as jnp
from jax import lax
from jax.experimental import pallas as pl
from jax.experimental.pallas import tpu as pltpu
from jax.experimental.pallas import tpu_sc as plsc

_LAT = 128
_HEAD = 4
_HDIM = 32
_NODES = 10000
_EDGES = 320000
_NC = 2
_NS = 16
_LH = _HEAD // _NC
_LLAT = _LH * _HDIM
_LW = _LLAT + 8
_C = 80
_G = _C // 16
_EPT = _EDGES // _NS
_CH = _EPT // _C
_NPAD = 10240
_RPT = _NPAD // _NS
_UNR = 8


def _qkv_body(emb_ref, qw_ref, kw_ref, vw_ref, q_ref, k_ref, v_ref):
    e = emb_ref[...]
    for w_ref, o_ref in ((qw_ref, q_ref), (kw_ref, k_ref), (vw_ref, v_ref)):
        full = jnp.dot(e, w_ref[...], preferred_element_type=jnp.float32)
        o_ref[0] = full[:, :_LLAT]
        o_ref[1] = full[:, _LLAT:]


def _qkv(embeds, qT, kT, vT):
    shp = jax.ShapeDtypeStruct((_NC, _NODES, _LLAT), jnp.float32)
    return pl.pallas_call(_qkv_body, out_shape=(shp, shp, shp))(embeds, qT, kT, vT)


def _sc_body(rows_h, cols_h, q_h, k_h, v_h, z_h, out_h,
             acc_s, ri0, ci0, qb0, kb0, vb0, wb0, ri1, ci1, qb1, kb1, vb1, wb1,
             ri2, ci2, ri3, ci3, sem, sem_i, sem_s):
    cid = lax.axis_index("c")
    sid = lax.axis_index("s")
    r0 = sid * _RPT
    qc, kc, vc = q_h.at[cid], k_h.at[cid], v_h.at[cid]
    cbase = sid * _CH
    pltpu.sync_copy(z_h.at[pl.ds(0, _RPT)], acc_s.at[pl.ds(r0, _RPT)])
    pltpu.sync_copy(z_h.at[pl.ds(0, _C)], wb0)
    pltpu.sync_copy(z_h.at[pl.ds(0, _C)], wb1)
    plsc.subcore_barrier()

    lanes = jnp.arange(16, dtype=jnp.int32)
    hvecs = [jnp.full((16,), h * _HDIM, jnp.int32) for h in range(_LH)]
    dbufs = ((qb0, kb0, vb0, wb0), (qb1, kb1, vb1, wb1))
    ibufs = ((ri0, ci0), (ri1, ci1), (ri2, ci2), (ri3, ci3))

    def issue_idx(ci, ri, cx):
        pltpu.async_copy(rows_h.at[pl.ds(cbase + ci, 1)], ri, sem_i)
        pltpu.async_copy(cols_h.at[pl.ds(cbase + ci, 1)], cx, sem_i)

    def drain_idx(ri, cx):
        pltpu.make_async_copy(rows_h.at[pl.ds(0, 1)], ri, sem_i).wait()
        pltpu.make_async_copy(rows_h.at[pl.ds(0, 1)], cx, sem_i).wait()

    def issue_gather(ri, cx, qb, kb, vb):
        pltpu.async_copy(qc.at[ri.at[0]], qb, sem)
        pltpu.async_copy(kc.at[cx.at[0]], kb, sem)
        pltpu.async_copy(vc.at[cx.at[0]], vb, sem)

    def drain_gather(qb, kb, vb):
        pltpu.make_async_copy(qc.at[pl.ds(0, _C)], qb, sem).wait()
        pltpu.make_async_copy(qc.at[pl.ds(0, _C)], kb, sem).wait()
        pltpu.make_async_copy(qc.at[pl.ds(0, _C)], vb, sem).wait()

    def issue_scatter(ri, wb):
        pltpu.async_copy(wb, acc_s.at[ri.at[0]], sem_s, add=True)

    def drain_scatter():
        pltpu.make_async_copy(wb0, acc_s.at[ri0.at[0]], sem_s).wait()

    def compute(qb, kb, vb, wb):
        def group(g, _):
            eids = (g % _G) * 16 + lanes
            zero = jnp.zeros((16,), jnp.float32)

            def att_d(dj, accs):
                accs = list(accs)
                for j in range(_UNR):
                    rot = (lanes + (dj * _UNR + j)) & 31
                    for h in range(_LH):
                        dvec = hvecs[h] + rot
                        qv = plsc.load_gather(qb, [eids, dvec])
                        kv = plsc.load_gather(kb, [eids, dvec])
                        accs[h] = accs[h] + qv * kv
                return tuple(accs)

            atts = lax.fori_loop(0, _HDIM // _UNR, att_d, (zero,) * _LH)
            exps = [jnp.exp(jnp.clip(a, -10.0, 10.0)) for a in atts]
            for h in range(_LH):
                plsc.store_scatter(
                    wb, [eids, jnp.full((16,), _LLAT + h, jnp.int32)], exps[h])

            def w_d(dj, _):
                for j in range(_UNR):
                    rot = (lanes + (dj * _UNR + j)) & 31
                    for h in range(_LH):
                        dvec = hvecs[h] + rot
                        vv = plsc.load_gather(vb, [eids, dvec])
                        plsc.store_scatter(wb, [eids, dvec], vv * exps[h])
                return 0

            lax.fori_loop(0, _HDIM // _UNR, w_d, 0)
            return 0

        lax.fori_loop(0, 2 * _G, group, 0)

    issue_idx(0, ri0, ci0)
    drain_idx(ri0, ci0)
    issue_gather(ri0, ci0, qb0, kb0, vb0)
    issue_idx(1, ri1, ci1)

    def body(ci_base, p, off):
        ci = ci_base
        qb, kb, vb, wb = dbufs[off % 2]
        nqb, nkb, nvb, _unused = dbufs[(off + 1) % 2]
        ri, cx = ibufs[off % 4]
        nri, ncx = ibufs[(off + 1) % 4]
        pri, pcx = ibufs[(off + 2) % 4]
        drain_idx(nri, ncx)
        issue_gather(nri, ncx, nqb, nkb, nvb)
        drain_gather(qb, kb, vb)
        @pl.when(ci >= 2)
        def _():
            drain_scatter()
        issue_idx(ci + 2, pri, pcx)
        compute(qb, kb, vb, wb)
        issue_scatter(ri, wb)

    def quad(p, _):
        for off in range(4):
            body(4 * p + off, p, off)
        return 0

    lax.fori_loop(0, (_CH - 2) // 4, quad, 0)
    drain_idx(ri1, ci1)
    issue_gather(ri1, ci1, qb1, kb1, vb1)
    drain_gather(qb0, kb0, vb0)
    drain_scatter()
    compute(qb0, kb0, vb0, wb0)
    issue_scatter(ri0, wb0)
    drain_gather(qb1, kb1, vb1)
    drain_scatter()
    compute(qb1, kb1, vb1, wb1)
    issue_scatter(ri1, wb1)
    drain_scatter()
    drain_scatter()
    plsc.subcore_barrier()
    pltpu.sync_copy(acc_s.at[pl.ds(r0, _RPT)], out_h.at[cid, pl.ds(r0, _RPT)])


_SC_MESH = plsc.VectorSubcoreMesh(
    core_axis_name="c", subcore_axis_name="s", num_cores=_NC, num_subcores=_NS)

_edge_call = pl.kernel(
    _sc_body,
    out_type=jax.ShapeDtypeStruct((_NC, _NPAD, _LW), jnp.float32),
    mesh=_SC_MESH,
    scratch_types=[
        pltpu.VMEM_SHARED((_NPAD, _LW), jnp.float32),
        pltpu.VMEM((1, _C), jnp.int32),
        pltpu.VMEM((1, _C), jnp.int32),
        pltpu.VMEM((_C, _LLAT), jnp.float32),
        pltpu.VMEM((_C, _LLAT), jnp.float32),
        pltpu.VMEM((_C, _LLAT), jnp.float32),
        pltpu.VMEM((_C, _LW), jnp.float32),
        pltpu.VMEM((1, _C), jnp.int32),
        pltpu.VMEM((1, _C), jnp.int32),
        pltpu.VMEM((_C, _LLAT), jnp.float32),
        pltpu.VMEM((_C, _LLAT), jnp.float32),
        pltpu.VMEM((_C, _LLAT), jnp.float32),
        pltpu.VMEM((_C, _LW), jnp.float32),
        pltpu.VMEM((1, _C), jnp.int32),
        pltpu.VMEM((1, _C), jnp.int32),
        pltpu.VMEM((1, _C), jnp.int32),
        pltpu.VMEM((1, _C), jnp.int32),
        pltpu.SemaphoreType.DMA,
        pltpu.SemaphoreType.DMA,
        pltpu.SemaphoreType.DMA,
    ],
    compiler_params=pltpu.CompilerParams(
        needs_layout_passes=False, use_tc_tiling_on_sc=False),
)


def _fin_body(acc_ref, sel_ref, out_ref):
    a0 = acc_ref[0, :_NODES]
    a1 = acc_ref[1, :_NODES]
    num = jnp.concatenate([a0[:, :_LLAT], a1[:, :_LLAT]], axis=1)
    norm = jnp.concatenate(
        [a0[:, _LLAT:_LLAT + _LH], a1[:, _LLAT:_LLAT + _LH]], axis=1)
    den = jnp.dot(norm, sel_ref[...], preferred_element_type=jnp.float32)
    out_ref[...] = num / (den + 1e-8)


def _finalize(acc):
    sel = jnp.repeat(jnp.eye(_HEAD, dtype=jnp.float32), _HDIM, axis=1)
    return pl.pallas_call(
        _fin_body,
        out_shape=jax.ShapeDtypeStruct((_NODES, _LAT), jnp.float32),
    )(acc, sel)


def kernel(edge_index, embeds, qTrans, kTrans, vTrans):
    rows = edge_index[0].reshape(_NS * _CH, _C)
    cols = edge_index[1].reshape(_NS * _CH, _C)
    q, k, v = _qkv(embeds, qTrans, kTrans, vTrans)
    zeros = jnp.zeros((_RPT, _LW), jnp.float32)
    acc = _edge_call(rows, cols, q, k, v, zeros)
    return _finalize(acc)


# speedup vs baseline: 1.0999x; 1.0999x over previous
import jax
import jax.numpy as jnp
from jax import lax
from jax.experimental import pallas as pl
from jax.experimental.pallas import tpu as pltpu
from jax.experimental.pallas import tpu_sc as plsc

_LAT = 128
_HEAD = 4
_HDIM = 32
_NODES = 10000
_EDGES = 320000
_NC = 2
_NS = 16
_LH = _HEAD // _NC
_LLAT = _LH * _HDIM
_LW = _LLAT + 8
_C = 80
_G = _C // 16
_EPT = _EDGES // _NS
_CH = _EPT // _C
_NPAD = 10240
_RPT = _NPAD // _NS
_UNR = 8


def _qkv_body(emb_ref, qw_ref, kw_ref, vw_ref, q_ref, k_ref, v_ref):
    e = emb_ref[...]
    for w_ref, o_ref in ((qw_ref, q_ref), (kw_ref, k_ref), (vw_ref, v_ref)):
        full = jnp.dot(e, w_ref[...], preferred_element_type=jnp.float32)
        o_ref[0] = full[:, :_LLAT]
        o_ref[1] = full[:, _LLAT:]


def _qkv(embeds, qT, kT, vT):
    shp = jax.ShapeDtypeStruct((_NC, _NODES, _LLAT), jnp.float32)
    return pl.pallas_call(_qkv_body, out_shape=(shp, shp, shp))(embeds, qT, kT, vT)


def _sc_body(rows_h, cols_h, q_h, k_h, v_h, z_h, out_h,
             acc_s, ri0, ci0, qb0, kb0, vb0, wb0, ri1, ci1, qb1, kb1, vb1, wb1,
             ri2, ci2, ri3, ci3, sem, sem_i, sem_s):
    cid = lax.axis_index("c")
    sid = lax.axis_index("s")
    r0 = sid * _RPT
    qc, kc, vc = q_h.at[cid], k_h.at[cid], v_h.at[cid]
    cbase = sid * _CH
    pltpu.sync_copy(z_h.at[pl.ds(0, _RPT)], acc_s.at[pl.ds(r0, _RPT)])
    pltpu.sync_copy(z_h.at[pl.ds(0, _C)], wb0)
    pltpu.sync_copy(z_h.at[pl.ds(0, _C)], wb1)
    plsc.subcore_barrier()

    lanes = jnp.arange(16, dtype=jnp.int32)
    hvecs = [jnp.full((16,), h * _HDIM, jnp.int32) for h in range(_LH)]
    dbufs = ((qb0, kb0, vb0, wb0), (qb1, kb1, vb1, wb1))
    ibufs = ((ri0, ci0), (ri1, ci1), (ri2, ci2), (ri3, ci3))

    def issue_idx(ci, ri, cx):
        pltpu.async_copy(rows_h.at[pl.ds(cbase + ci, 1)], ri, sem_i)
        pltpu.async_copy(cols_h.at[pl.ds(cbase + ci, 1)], cx, sem_i)

    def drain_idx(ri, cx):
        pltpu.make_async_copy(rows_h.at[pl.ds(0, 1)], ri, sem_i).wait()
        pltpu.make_async_copy(rows_h.at[pl.ds(0, 1)], cx, sem_i).wait()

    def issue_gather(ri, cx, qb, kb, vb):
        pltpu.async_copy(qc.at[ri.at[0]], qb, sem)
        pltpu.async_copy(kc.at[cx.at[0]], kb, sem)
        pltpu.async_copy(vc.at[cx.at[0]], vb, sem)

    def drain_gather(qb, kb, vb):
        pltpu.make_async_copy(qc.at[pl.ds(0, _C)], qb, sem).wait()
        pltpu.make_async_copy(qc.at[pl.ds(0, _C)], kb, sem).wait()
        pltpu.make_async_copy(qc.at[pl.ds(0, _C)], vb, sem).wait()

    def issue_scatter(ri, wb):
        pltpu.async_copy(wb, acc_s.at[ri.at[0]], sem_s, add=True)

    def drain_scatter():
        pltpu.make_async_copy(wb0, acc_s.at[ri0.at[0]], sem_s).wait()

    def compute(qb, kb, vb, wb):
        def group(g, _):
            eids = g * 16 + lanes
            zero = jnp.zeros((16,), jnp.float32)

            def att_d(dj, accs):
                accs = list(accs)
                for j in range(_UNR):
                    rot = (lanes + (dj * _UNR + j)) & 31
                    for h in range(_LH):
                        dvec = hvecs[h] + rot
                        qv = plsc.load_gather(qb, [eids, dvec])
                        kv = plsc.load_gather(kb, [eids, dvec])
                        accs[h] = accs[h] + qv * kv
                return tuple(accs)

            atts = lax.fori_loop(0, _HDIM // _UNR, att_d, (zero,) * _LH)
            exps = [jnp.exp(jnp.clip(a, -10.0, 10.0)) for a in atts]
            for h in range(_LH):
                plsc.store_scatter(
                    wb, [eids, jnp.full((16,), _LLAT + h, jnp.int32)], exps[h])

            def w_d(dj, _):
                for j in range(_UNR):
                    rot = (lanes + ((dj % (_HDIM // _UNR)) * _UNR + j)) & 31
                    for h in range(_LH):
                        dvec = hvecs[h] + rot
                        vv = plsc.load_gather(vb, [eids, dvec])
                        plsc.store_scatter(wb, [eids, dvec], vv * exps[h])
                return 0

            lax.fori_loop(0, 2 * (_HDIM // _UNR), w_d, 0)
            return 0

        lax.fori_loop(0, _G, group, 0)

    issue_idx(0, ri0, ci0)
    drain_idx(ri0, ci0)
    issue_gather(ri0, ci0, qb0, kb0, vb0)
    issue_idx(1, ri1, ci1)

    def body(ci_base, p, off):
        ci = ci_base
        qb, kb, vb, wb = dbufs[off % 2]
        nqb, nkb, nvb, _unused = dbufs[(off + 1) % 2]
        ri, cx = ibufs[off % 4]
        nri, ncx = ibufs[(off + 1) % 4]
        pri, pcx = ibufs[(off + 2) % 4]
        drain_idx(nri, ncx)
        issue_gather(nri, ncx, nqb, nkb, nvb)
        drain_gather(qb, kb, vb)
        @pl.when(ci >= 2)
        def _():
            drain_scatter()
        issue_idx(ci + 2, pri, pcx)
        compute(qb, kb, vb, wb)
        issue_scatter(ri, wb)

    def quad(p, _):
        for off in range(4):
            body(4 * p + off, p, off)
        return 0

    lax.fori_loop(0, (_CH - 2) // 4, quad, 0)
    drain_idx(ri1, ci1)
    issue_gather(ri1, ci1, qb1, kb1, vb1)
    drain_gather(qb0, kb0, vb0)
    drain_scatter()
    compute(qb0, kb0, vb0, wb0)
    issue_scatter(ri0, wb0)
    drain_gather(qb1, kb1, vb1)
    drain_scatter()
    compute(qb1, kb1, vb1, wb1)
    issue_scatter(ri1, wb1)
    drain_scatter()
    drain_scatter()
    plsc.subcore_barrier()
    pltpu.sync_copy(acc_s.at[pl.ds(r0, _RPT)], out_h.at[cid, pl.ds(r0, _RPT)])


_SC_MESH = plsc.VectorSubcoreMesh(
    core_axis_name="c", subcore_axis_name="s", num_cores=_NC, num_subcores=_NS)

_edge_call = pl.kernel(
    _sc_body,
    out_type=jax.ShapeDtypeStruct((_NC, _NPAD, _LW), jnp.float32),
    mesh=_SC_MESH,
    scratch_types=[
        pltpu.VMEM_SHARED((_NPAD, _LW), jnp.float32),
        pltpu.VMEM((1, _C), jnp.int32),
        pltpu.VMEM((1, _C), jnp.int32),
        pltpu.VMEM((_C, _LLAT), jnp.float32),
        pltpu.VMEM((_C, _LLAT), jnp.float32),
        pltpu.VMEM((_C, _LLAT), jnp.float32),
        pltpu.VMEM((_C, _LW), jnp.float32),
        pltpu.VMEM((1, _C), jnp.int32),
        pltpu.VMEM((1, _C), jnp.int32),
        pltpu.VMEM((_C, _LLAT), jnp.float32),
        pltpu.VMEM((_C, _LLAT), jnp.float32),
        pltpu.VMEM((_C, _LLAT), jnp.float32),
        pltpu.VMEM((_C, _LW), jnp.float32),
        pltpu.VMEM((1, _C), jnp.int32),
        pltpu.VMEM((1, _C), jnp.int32),
        pltpu.VMEM((1, _C), jnp.int32),
        pltpu.VMEM((1, _C), jnp.int32),
        pltpu.SemaphoreType.DMA,
        pltpu.SemaphoreType.DMA,
        pltpu.SemaphoreType.DMA,
    ],
    compiler_params=pltpu.CompilerParams(
        needs_layout_passes=False, use_tc_tiling_on_sc=False),
)


def _fin_body(acc_ref, sel_ref, out_ref):
    a0 = acc_ref[0, :_NODES]
    a1 = acc_ref[1, :_NODES]
    num = jnp.concatenate([a0[:, :_LLAT], a1[:, :_LLAT]], axis=1)
    norm = jnp.concatenate(
        [a0[:, _LLAT:_LLAT + _LH], a1[:, _LLAT:_LLAT + _LH]], axis=1)
    den = jnp.dot(norm, sel_ref[...], preferred_element_type=jnp.float32)
    out_ref[...] = num / (den + 1e-8)


def _finalize(acc):
    sel = jnp.repeat(jnp.eye(_HEAD, dtype=jnp.float32), _HDIM, axis=1)
    return pl.pallas_call(
        _fin_body,
        out_shape=jax.ShapeDtypeStruct((_NODES, _LAT), jnp.float32),
    )(acc, sel)


def kernel(edge_index, embeds, qTrans, kTrans, vTrans):
    rows = edge_index[0].reshape(_NS * _CH, _C)
    cols = edge_index[1].reshape(_NS * _CH, _C)
    q, k, v = _qkv(embeds, qTrans, kTrans, vTrans)
    zeros = jnp.zeros((_RPT, _LW), jnp.float32)
    acc = _edge_call(rows, cols, q, k, v, zeros)
    return _finalize(acc)


# speedup vs baseline: 2.8129x; 2.5573x over previous
import jax
import jax.numpy as jnp
from jax import lax
from jax.experimental import pallas as pl
from jax.experimental.pallas import tpu as pltpu
from jax.experimental.pallas import tpu_sc as plsc

_LAT = 128
_HEAD = 4
_HDIM = 32
_NODES = 10000
_EDGES = 320000
_NC = 2
_NS = 16
_LH = _HEAD // _NC
_LLAT = _LH * _HDIM
_LW = _LLAT + 8
_C = 80
_G = _C // 16
_EPT = _EDGES // _NS
_CH = _EPT // _C
_NPAD = 10240
_RPT = _NPAD // _NS
_UNR = 8


def _qkv_body(emb_ref, qw_ref, kw_ref, vw_ref, q_ref, k_ref, v_ref):
    e = emb_ref[...]
    for w_ref, o_ref in ((qw_ref, q_ref), (kw_ref, k_ref), (vw_ref, v_ref)):
        full = jnp.dot(e, w_ref[...], preferred_element_type=jnp.float32)
        o_ref[0] = full[:, :_LLAT]
        o_ref[1] = full[:, _LLAT:]


def _qkv(embeds, qT, kT, vT):
    shp = jax.ShapeDtypeStruct((_NC, _NODES, _LLAT), jnp.float32)
    return pl.pallas_call(_qkv_body, out_shape=(shp, shp, shp))(embeds, qT, kT, vT)


def _sc_body(rows_h, cols_h, q_h, k_h, v_h, z_h, out_h,
             acc_s, ri0, ci0, qb0, kb0, vb0, wb0, ri1, ci1, qb1, kb1, vb1, wb1,
             ri2, ci2, ri3, ci3, sem, sem_i, sem_s):
    cid = lax.axis_index("c")
    sid = lax.axis_index("s")
    r0 = sid * _RPT
    qc, kc, vc = q_h.at[cid], k_h.at[cid], v_h.at[cid]
    cbase = sid * _CH
    pltpu.sync_copy(z_h.at[pl.ds(0, _RPT)], acc_s.at[pl.ds(r0, _RPT)])
    pltpu.sync_copy(z_h.at[pl.ds(0, _C)], wb0)
    pltpu.sync_copy(z_h.at[pl.ds(0, _C)], wb1)
    plsc.subcore_barrier()

    lanes = jnp.arange(16, dtype=jnp.int32)
    hvecs = [jnp.full((16,), h * _HDIM, jnp.int32) for h in range(_LH)]
    dbufs = ((qb0, kb0, vb0, wb0), (qb1, kb1, vb1, wb1))
    ibufs = ((ri0, ci0), (ri1, ci1), (ri2, ci2), (ri3, ci3))

    def issue_idx(ci, ri, cx):
        pltpu.async_copy(rows_h.at[pl.ds(cbase + ci, 1)], ri, sem_i)
        pltpu.async_copy(cols_h.at[pl.ds(cbase + ci, 1)], cx, sem_i)

    def drain_idx(ri, cx):
        pltpu.make_async_copy(rows_h.at[pl.ds(0, 1)], ri, sem_i).wait()
        pltpu.make_async_copy(rows_h.at[pl.ds(0, 1)], cx, sem_i).wait()

    def issue_gather(ri, cx, qb, kb, vb):
        pltpu.async_copy(qc.at[ri.at[0]], qb, sem)
        pltpu.async_copy(kc.at[cx.at[0]], kb, sem)
        pltpu.async_copy(vc.at[cx.at[0]], vb, sem)

    def drain_gather(qb, kb, vb):
        pltpu.make_async_copy(qc.at[pl.ds(0, _C)], qb, sem).wait()
        pltpu.make_async_copy(qc.at[pl.ds(0, _C)], kb, sem).wait()
        pltpu.make_async_copy(qc.at[pl.ds(0, _C)], vb, sem).wait()

    def issue_scatter(ri, wb):
        pltpu.async_copy(wb, acc_s.at[ri.at[0]], sem_s, add=True)

    def drain_scatter():
        pltpu.make_async_copy(wb0, acc_s.at[ri0.at[0]], sem_s).wait()

    def compute(qb, kb, vb, wb):
        def group(g, _):
            eids = g * 16 + lanes
            zero = jnp.zeros((16,), jnp.float32)

            def att_d(dj, accs):
                accs = list(accs)
                for j in range(_UNR):
                    rot = (lanes + (dj * _UNR + j)) & 31
                    for h in range(_LH):
                        dvec = hvecs[h] + rot
                        qv = plsc.load_gather(qb, [eids, dvec])
                        kv = plsc.load_gather(kb, [eids, dvec])
                        accs[h] = accs[h] + qv * kv
                return tuple(accs)

            atts = lax.fori_loop(0, _HDIM // _UNR, att_d, (zero,) * _LH)
            exps = [jnp.exp(jnp.clip(a, -10.0, 10.0)) for a in atts]
            for h in range(_LH):
                plsc.store_scatter(
                    wb, [eids, jnp.full((16,), _LLAT + h, jnp.int32)], exps[h])

            def w_d(dj, _):
                for j0 in range(0, _UNR, 4):
                    idxs, vals = [], []
                    for j in range(j0, j0 + 4):
                        rot = (lanes + (dj * _UNR + j)) & 31
                        for h in range(_LH):
                            dvec = hvecs[h] + rot
                            idxs.append(dvec)
                            vals.append(
                                plsc.load_gather(vb, [eids, dvec]) * exps[h])
                    for dvec, val in zip(idxs, vals):
                        plsc.store_scatter(wb, [eids, dvec], val)
                return 0

            lax.fori_loop(0, _HDIM // _UNR, w_d, 0)
            return 0

        lax.fori_loop(0, _G, group, 0)

    issue_idx(0, ri0, ci0)
    drain_idx(ri0, ci0)
    issue_gather(ri0, ci0, qb0, kb0, vb0)
    issue_idx(1, ri1, ci1)

    def body(ci_base, p, off):
        ci = ci_base
        qb, kb, vb, wb = dbufs[off % 2]
        nqb, nkb, nvb, _unused = dbufs[(off + 1) % 2]
        ri, cx = ibufs[off % 4]
        nri, ncx = ibufs[(off + 1) % 4]
        pri, pcx = ibufs[(off + 2) % 4]
        drain_idx(nri, ncx)
        issue_gather(nri, ncx, nqb, nkb, nvb)
        drain_gather(qb, kb, vb)
        @pl.when(ci >= 2)
        def _():
            drain_scatter()
        issue_idx(ci + 2, pri, pcx)
        compute(qb, kb, vb, wb)
        issue_scatter(ri, wb)

    def quad(p, _):
        for off in range(4):
            body(4 * p + off, p, off)
        return 0

    lax.fori_loop(0, (_CH - 2) // 4, quad, 0)
    drain_idx(ri1, ci1)
    issue_gather(ri1, ci1, qb1, kb1, vb1)
    drain_gather(qb0, kb0, vb0)
    drain_scatter()
    compute(qb0, kb0, vb0, wb0)
    issue_scatter(ri0, wb0)
    drain_gather(qb1, kb1, vb1)
    drain_scatter()
    compute(qb1, kb1, vb1, wb1)
    issue_scatter(ri1, wb1)
    drain_scatter()
    drain_scatter()
    plsc.subcore_barrier()
    pltpu.sync_copy(acc_s.at[pl.ds(r0, _RPT)], out_h.at[cid, pl.ds(r0, _RPT)])


_SC_MESH = plsc.VectorSubcoreMesh(
    core_axis_name="c", subcore_axis_name="s", num_cores=_NC, num_subcores=_NS)

_edge_call = pl.kernel(
    _sc_body,
    out_type=jax.ShapeDtypeStruct((_NC, _NPAD, _LW), jnp.float32),
    mesh=_SC_MESH,
    scratch_types=[
        pltpu.VMEM_SHARED((_NPAD, _LW), jnp.float32),
        pltpu.VMEM((1, _C), jnp.int32),
        pltpu.VMEM((1, _C), jnp.int32),
        pltpu.VMEM((_C, _LLAT), jnp.float32),
        pltpu.VMEM((_C, _LLAT), jnp.float32),
        pltpu.VMEM((_C, _LLAT), jnp.float32),
        pltpu.VMEM((_C, _LW), jnp.float32),
        pltpu.VMEM((1, _C), jnp.int32),
        pltpu.VMEM((1, _C), jnp.int32),
        pltpu.VMEM((_C, _LLAT), jnp.float32),
        pltpu.VMEM((_C, _LLAT), jnp.float32),
        pltpu.VMEM((_C, _LLAT), jnp.float32),
        pltpu.VMEM((_C, _LW), jnp.float32),
        pltpu.VMEM((1, _C), jnp.int32),
        pltpu.VMEM((1, _C), jnp.int32),
        pltpu.VMEM((1, _C), jnp.int32),
        pltpu.VMEM((1, _C), jnp.int32),
        pltpu.SemaphoreType.DMA,
        pltpu.SemaphoreType.DMA,
        pltpu.SemaphoreType.DMA,
    ],
    compiler_params=pltpu.CompilerParams(
        needs_layout_passes=False, use_tc_tiling_on_sc=False),
)


def _fin_body(acc_ref, sel_ref, out_ref):
    a0 = acc_ref[0, :_NODES]
    a1 = acc_ref[1, :_NODES]
    num = jnp.concatenate([a0[:, :_LLAT], a1[:, :_LLAT]], axis=1)
    norm = jnp.concatenate(
        [a0[:, _LLAT:_LLAT + _LH], a1[:, _LLAT:_LLAT + _LH]], axis=1)
    den = jnp.dot(norm, sel_ref[...], preferred_element_type=jnp.float32)
    out_ref[...] = num / (den + 1e-8)


def _finalize(acc):
    sel = jnp.repeat(jnp.eye(_HEAD, dtype=jnp.float32), _HDIM, axis=1)
    return pl.pallas_call(
        _fin_body,
        out_shape=jax.ShapeDtypeStruct((_NODES, _LAT), jnp.float32),
    )(acc, sel)


def kernel(edge_index, embeds, qTrans, kTrans, vTrans):
    rows = edge_index[0].reshape(_NS * _CH, _C)
    cols = edge_index[1].reshape(_NS * _CH, _C)
    q, k, v = _qkv(embeds, qTrans, kTrans, vTrans)
    zeros = jnp.zeros((_RPT, _LW), jnp.float32)
    acc = _edge_call(rows, cols, q, k, v, zeros)
    return _finalize(acc)
